# baseline (speedup 1.0000x reference)
import functools
import math

import jax
import jax.numpy as jnp
from jax import lax
from jax.experimental import pallas as pl
from jax.experimental.pallas import tpu as pltpu

B = 2
C = 80
IMG = 800.0
SCORE_THRESH = 0.05
NMS_THRESH = 0.5
DETS = 100
BBOX_CLAMP = 4.135166556742356
K_ANC = 128
T_LOGIT = -math.log((1.0 - SCORE_THRESH) / SCORE_THRESH)


def _amax_body(n, a_blk, x_ref, o_ref):
    i = pl.program_id(1)
    x = x_ref[0]
    xm = jnp.max(jnp.where(x > T_LOGIT, x, -1e30), axis=1)
    m = jnp.where(xm > -1e29, 1.0 / (1.0 + jnp.exp(-xm)), -1.0)
    rows = i * a_blk + lax.broadcasted_iota(jnp.int32, (a_blk,), 0)
    m = jnp.where(rows < n, m, -2.0)
    o_ref[0, 0] = m


def _anchor_max(x, a_blk):
    n = x.shape[1]
    nb = pl.cdiv(n, a_blk)
    out = pl.pallas_call(
        functools.partial(_amax_body, n, a_blk),
        grid=(B, nb),
        in_specs=[pl.BlockSpec((1, a_blk, C), lambda b, i: (b, i, 0))],
        out_specs=pl.BlockSpec((1, 1, a_blk), lambda b, i: (b * nb + i, 0, 0)),
        out_shape=jax.ShapeDtypeStruct((B * nb, 1, a_blk), jnp.float32),
    )(x)
    return out.reshape(B, nb * a_blk)


_KEY_NEG1 = -1065353217
_KEY_005 = 1028443341
_KEY_HI = 1065353216
_BISECT_ITERS = 26


def _keys_of(m):
    bi = lax.bitcast_convert_type(m, jnp.int32)
    return jnp.where(bi >= 0, bi, jnp.int32(-2147483648) ^ (~bi))


def _select_body(rows_list, *refs):
    m_refs, idx_ref, kscr = refs[:4], refs[4], refs[5]
    lane = lax.broadcasted_iota(jnp.int32, (1, 128), 1)
    for l, rows in enumerate(rows_list):
        for b in range(B):
            kscr[0:rows, :] = _keys_of(m_refs[l][b, 0:rows, :])

            def cnt(t):
                return jnp.sum((kscr[0:rows, :] > t).astype(jnp.int32))

            g05 = cnt(_KEY_005 - 1)

            def bis(_, c):
                lo, hi = c
                mid = lo + (hi - lo) // 2
                big = cnt(mid) >= K_ANC
                return jnp.where(big, mid, lo), jnp.where(big, hi, mid)

            lo, hi = lax.fori_loop(0, _BISECT_ITERS, bis,
                                   (jnp.int32(_KEY_005 - 1), jnp.int32(_KEY_HI)))
            v = jnp.where(g05 >= K_ANC, hi, jnp.int32(_KEY_NEG1))
            g = cnt(v)

            def ext_body(c):
                ridx, jprev, taken, bud = c
                krow = kscr[pl.ds(ridx, 1), :]
                rem = ((krow > v) | ((krow == v) & (bud > 0))) & (lane > jprev)
                has = jnp.any(rem)
                j = jnp.min(jnp.where(rem, lane, 128))
                is_tie = jnp.sum(jnp.where(rem & (lane == j),
                                           (krow == v).astype(jnp.int32), 0))

                @pl.when(has)
                def _():
                    idx_ref[b, l, taken] = ridx * 128 + j

                return (jnp.where(has, ridx, ridx + 1),
                        jnp.where(has, j, -1),
                        taken + has.astype(jnp.int32),
                        bud - jnp.where(has, is_tie, 0))

            lax.while_loop(lambda c: c[2] < K_ANC, ext_body,
                           (jnp.int32(0), jnp.int32(-1), jnp.int32(0),
                            K_ANC - g))


def _select(m_list):
    rows_list = [m.shape[1] // 128 for m in m_list]
    ms = [m.reshape(B, r, 128) for m, r in zip(m_list, rows_list)]
    return pl.pallas_call(
        functools.partial(_select_body, rows_list),
        in_specs=[pl.BlockSpec(memory_space=pltpu.VMEM) for _ in ms],
        out_specs=pl.BlockSpec(memory_space=pltpu.SMEM),
        out_shape=jax.ShapeDtypeStruct((B, 4, K_ANC), jnp.int32),
        scratch_shapes=[pltpu.VMEM((max(rows_list), 128), jnp.int32)],
    )(*ms)


def _decode(rel, anc):
    w = anc[:, 2] - anc[:, 0]
    h = anc[:, 3] - anc[:, 1]
    cx = anc[:, 0] + 0.5 * w
    cy = anc[:, 1] + 0.5 * h
    dx, dy = rel[:, 0], rel[:, 1]
    dw = jnp.minimum(rel[:, 2], BBOX_CLAMP)
    dh = jnp.minimum(rel[:, 3], BBOX_CLAMP)
    pcx = dx * w + cx
    pcy = dy * h + cy
    pw = jnp.exp(dw) * w
    ph = jnp.exp(dh) * h
    return jnp.stack([pcx - 0.5 * pw, pcy - 0.5 * ph,
                      pcx + 0.5 * pw, pcy + 0.5 * ph], axis=1)


def _nms_body(bx_ref, sc_ref, lb_ref, ob_ref, os_ref, ol_ref):
    ncand = sc_ref.shape[2]
    x1 = bx_ref[0, 0:1, :]
    y1 = bx_ref[0, 1:2, :]
    x2 = bx_ref[0, 2:3, :]
    y2 = bx_ref[0, 3:4, :]
    scores = sc_ref[0]
    labf = lb_ref[0].astype(jnp.float32)
    offs = labf * (IMG + 1.0)
    nx1, ny1, nx2, ny2 = x1 + offs, y1 + offs, x2 + offs, y2 + offs
    area = (nx2 - nx1) * (ny2 - ny1)
    iota = lax.broadcasted_iota(jnp.int32, (1, ncand), 1)
    kiota = lax.broadcasted_iota(jnp.int32, (1, DETS), 1)
    zrow = jnp.zeros((1, DETS), jnp.float32)

    def step(i, carry):
        work, fs, fl, b1, b2, b3, b4 = carry
        mx = jnp.max(work)
        j = jnp.min(jnp.where(work == mx, iota, ncand))
        jm = iota == j

        def ext(row):
            return jnp.sum(jnp.where(jm, row, 0.0))

        jx1, jy1, jx2, jy2 = ext(nx1), ext(ny1), ext(nx2), ext(ny2)
        ja = (jx2 - jx1) * (jy2 - jy1)
        inter = (jnp.maximum(jnp.minimum(jx2, nx2) - jnp.maximum(jx1, nx1), 0.0)
                 * jnp.maximum(jnp.minimum(jy2, ny2) - jnp.maximum(jy1, ny1), 0.0))
        iou = inter / (ja + area - inter + 1e-9)
        im = kiota == i
        fs = jnp.where(im, ext(scores), fs)
        fl = jnp.where(im, ext(labf), fl)
        b1 = jnp.where(im, ext(x1), b1)
        b2 = jnp.where(im, ext(y1), b2)
        b3 = jnp.where(im, ext(x2), b3)
        b4 = jnp.where(im, ext(y2), b4)
        work = jnp.where(iou > NMS_THRESH, -jnp.inf, work)
        work = jnp.where(jm, -jnp.inf, work)
        return work, fs, fl, b1, b2, b3, b4

    _, fs, fl, b1, b2, b3, b4 = lax.fori_loop(
        0, DETS, step, (scores, zrow, zrow, zrow, zrow, zrow, zrow))

    valid = fs > SCORE_THRESH
    vf = valid.astype(jnp.float32)
    os_ref[0] = jnp.where(valid, fs, 0.0)
    ol_ref[0] = jnp.where(valid, fl, 0.0).astype(jnp.int32)
    ob_ref[0, 0:1, :] = b1 * vf
    ob_ref[0, 1:2, :] = b2 * vf
    ob_ref[0, 2:3, :] = b3 * vf
    ob_ref[0, 3:4, :] = b4 * vf


def _nms(boxes_t, scores, labels):
    ncand = scores.shape[1]
    ob, os_, ol = pl.pallas_call(
        _nms_body,
        grid=(B,),
        in_specs=[
            pl.BlockSpec((1, 4, ncand), lambda b: (b, 0, 0)),
            pl.BlockSpec((1, 1, ncand), lambda b: (b, 0, 0)),
            pl.BlockSpec((1, 1, ncand), lambda b: (b, 0, 0)),
        ],
        out_specs=[
            pl.BlockSpec((1, 4, DETS), lambda b: (b, 0, 0)),
            pl.BlockSpec((1, 1, DETS), lambda b: (b, 0, 0)),
            pl.BlockSpec((1, 1, DETS), lambda b: (b, 0, 0)),
        ],
        out_shape=[
            jax.ShapeDtypeStruct((B, 4, DETS), jnp.float32),
            jax.ShapeDtypeStruct((B, 1, DETS), jnp.float32),
            jax.ShapeDtypeStruct((B, 1, DETS), jnp.int32),
        ],
    )(boxes_t, scores[:, None, :], labels[:, None, :])
    return ob, os_[:, 0, :], ol[:, 0, :]


def kernel(cls_logits_l0, cls_logits_l1, cls_logits_l2, cls_logits_l3,
           bbox_reg_l0, bbox_reg_l1, bbox_reg_l2, bbox_reg_l3,
           anchors_l0, anchors_l1, anchors_l2, anchors_l3):
    logits = [cls_logits_l0, cls_logits_l1, cls_logits_l2, cls_logits_l3]
    regs = [bbox_reg_l0, bbox_reg_l1, bbox_reg_l2, bbox_reg_l3]
    ancs = [anchors_l0, anchors_l1, anchors_l2, anchors_l3]
    blks = [2048, 2048, 2048, 1536]

    m_list = [_anchor_max(logits[l], blks[l]) for l in range(4)]
    idx = _select(m_list)

    outs = []
    for b in range(B):
        all_b, all_s, all_l = [], [], []
        for l in range(4):
            a_sel = idx[b, l]
            glog = logits[l][b][a_sel]
            s = jax.nn.sigmoid(glog)
            sc = jnp.where(s > SCORE_THRESH, s, -1.0).reshape(-1)
            top_s, top_i = lax.top_k(sc, DETS)
            a_idx = a_sel[top_i // C]
            labels = top_i % C
            boxes = _decode(regs[l][b][a_idx], ancs[l][a_idx])
            boxes = jnp.clip(boxes, 0.0, IMG)
            all_b.append(boxes)
            all_s.append(top_s)
            all_l.append(labels)
        outs.append((jnp.concatenate(all_b, axis=0),
                     jnp.concatenate(all_s, axis=0),
                     jnp.concatenate(all_l, axis=0)))
    boxes_t = jnp.stack([o[0].T for o in outs], axis=0)
    scores = jnp.stack([o[1] for o in outs], axis=0)
    labels = jnp.stack([o[2] for o in outs], axis=0)
    ob, os_, ol = _nms(boxes_t, scores, labels)
    return jnp.swapaxes(ob, 1, 2), os_, ol


# speedup vs baseline: 2.3432x; 2.3432x over previous
import functools
import math

import jax
import jax.numpy as jnp
from jax import lax
from jax.experimental import pallas as pl
from jax.experimental.pallas import tpu as pltpu

B = 2
C = 80
IMG = 800.0
SCORE_THRESH = 0.05
NMS_THRESH = 0.5
DETS = 100
BBOX_CLAMP = 4.135166556742356
K_ANC = 128
T_LOGIT = -math.log((1.0 - SCORE_THRESH) / SCORE_THRESH)


def _amax_body(n, a_blk, x_ref, o_ref):
    i = pl.program_id(1)
    x = x_ref[0]
    xm = jnp.max(jnp.where(x > T_LOGIT, x, -1e30), axis=1)
    m = jnp.where(xm > -1e29, 1.0 / (1.0 + jnp.exp(-xm)), -1.0)
    rows = i * a_blk + lax.broadcasted_iota(jnp.int32, (a_blk,), 0)
    m = jnp.where(rows < n, m, -2.0)
    o_ref[0, 0] = m


def _anchor_max(x, a_blk):
    n = x.shape[1]
    nb = pl.cdiv(n, a_blk)
    out = pl.pallas_call(
        functools.partial(_amax_body, n, a_blk),
        grid=(B, nb),
        in_specs=[pl.BlockSpec((1, a_blk, C), lambda b, i: (b, i, 0))],
        out_specs=pl.BlockSpec((1, 1, a_blk), lambda b, i: (b * nb + i, 0, 0)),
        out_shape=jax.ShapeDtypeStruct((B * nb, 1, a_blk), jnp.float32),
    )(x)
    return out.reshape(B, nb * a_blk)


_KEY_NEG1 = -1065353217
_KEY_005 = 1028443341
_KEY_HI = 1065353216
_BISECT_ITERS = 26


def _keys_of(m):
    bi = lax.bitcast_convert_type(m, jnp.int32)
    return jnp.where(bi >= 0, bi, jnp.int32(-2147483648) ^ (~bi))


def _tr(x, eye):
    if x.shape[1] == 1:
        return lax.dot_general(x, eye, (((0,), (0,)), ((), ())),
                               preferred_element_type=jnp.float32)
    return lax.dot_general(eye, x, (((1,), (1,)), ((), ())),
                           preferred_element_type=jnp.float32)


def _dot(a, b):
    return jnp.dot(a, b, preferred_element_type=jnp.float32)


def _select_body(rows_list, *refs):
    m_refs, idx_ref, kscr = refs[:4], refs[4], refs[5]
    offs, o = [], 0
    for r in rows_list:
        offs.extend([o, o + r])
        o += 2 * r
    tasks = [(l, b, rows_list[l], offs[2 * l + b])
             for l in range(4) for b in range(B)]
    for l, b, rows, off in tasks:
        kscr[pl.ds(off, rows), :] = _keys_of(m_refs[l][b])

    def cnt(off, rows, t):
        return jnp.sum((kscr[pl.ds(off, rows), :] > t).astype(jnp.int32))

    g05s = [cnt(off, rows, _KEY_005 - 1) for _, _, rows, off in tasks]

    def bis(_, c):
        los, his = c
        nlo, nhi = [], []
        for (l, b, rows, off), lo, hi in zip(tasks, los, his):
            mid = lo + (hi - lo) // 2
            big = cnt(off, rows, mid) >= K_ANC
            nlo.append(jnp.where(big, mid, lo))
            nhi.append(jnp.where(big, hi, mid))
        return tuple(nlo), tuple(nhi)

    init = (tuple(jnp.int32(_KEY_005 - 1) for _ in tasks),
            tuple(jnp.int32(_KEY_HI) for _ in tasks))
    _, his = lax.fori_loop(0, _BISECT_ITERS, bis, init)
    vs = [jnp.where(g05 >= K_ANC, hi, jnp.int32(_KEY_NEG1))
          for g05, hi in zip(g05s, his)]
    gs = [cnt(off, rows, v) for (_, _, rows, off), v in zip(tasks, vs)]

    ut128 = (lax.broadcasted_iota(jnp.int32, (128, 128), 0)
             < lax.broadcasted_iota(jnp.int32, (128, 128), 1)).astype(jnp.float32)
    lane_f = lax.broadcasted_iota(jnp.int32, (128, 128), 1).astype(jnp.float32)
    k_col = lax.broadcasted_iota(jnp.int32, (K_ANC, 1), 0).astype(jnp.float32)

    eyes, uts = {}, {}
    for r in set(rows_list):
        i0 = lax.broadcasted_iota(jnp.int32, (r, r), 0)
        i1 = lax.broadcasted_iota(jnp.int32, (r, r), 1)
        eyes[r] = (i0 == i1).astype(jnp.float32)
        uts[r] = (i0 < i1).astype(jnp.float32)

    for (l, b, rows, off), v, g in zip(tasks, vs, gs):
        keys = kscr[pl.ds(off, rows), :]
        strict = keys > v
        tie = keys == v
        tie_f = tie.astype(jnp.float32)
        p_tie = _dot(tie_f, ut128)
        tcnt_col = p_tie[:, 127:128] + tie_f[:, 127:128]
        tpref_row = _dot(_tr(tcnt_col, eyes[rows]), uts[rows])
        tie_rank = _tr(tpref_row, eyes[rows]) + p_tie
        g_f = g.astype(jnp.float32)
        sel = strict | (tie & (g_f + tie_rank < float(K_ANC)))
        sel_f = sel.astype(jnp.float32)
        p_sel = _dot(sel_f, ut128)
        scnt_col = p_sel[:, 127:128] + sel_f[:, 127:128]
        spref_row = _dot(_tr(scnt_col, eyes[rows]), uts[rows])
        rmask = (spref_row <= k_col).astype(jnp.float32)
        r_col = jnp.sum(rmask, axis=1, keepdims=True) - 1.0
        onehot_r = (lax.broadcasted_iota(jnp.int32, (K_ANC, rows), 1)
                    .astype(jnp.float32) == r_col).astype(jnp.float32)
        spref_at_k = _dot(onehot_r, _tr(spref_row, eyes[rows]))
        gmat = _dot(onehot_r, jnp.where(sel, p_sel, 1e9))
        q_col = k_col - spref_at_k
        c_col = jnp.sum(jnp.where(gmat == q_col, lane_f, 0.0),
                        axis=1, keepdims=True)
        idx_ref[b, l] = (r_col * 128.0 + c_col).astype(jnp.int32)


def _select(m_list):
    rows_list = [m.shape[1] // 128 for m in m_list]
    ms = [m.reshape(B, r, 128) for m, r in zip(m_list, rows_list)]
    total = 2 * sum(rows_list)
    out = pl.pallas_call(
        functools.partial(_select_body, rows_list),
        in_specs=[pl.BlockSpec(memory_space=pltpu.VMEM) for _ in ms],
        out_specs=pl.BlockSpec(memory_space=pltpu.VMEM),
        out_shape=jax.ShapeDtypeStruct((B, 4, K_ANC, 1), jnp.int32),
        scratch_shapes=[pltpu.VMEM((total, 128), jnp.int32)],
    )(*ms)
    return out.reshape(B, 4, K_ANC)


def _decode(rel, anc):
    w = anc[:, 2] - anc[:, 0]
    h = anc[:, 3] - anc[:, 1]
    cx = anc[:, 0] + 0.5 * w
    cy = anc[:, 1] + 0.5 * h
    dx, dy = rel[:, 0], rel[:, 1]
    dw = jnp.minimum(rel[:, 2], BBOX_CLAMP)
    dh = jnp.minimum(rel[:, 3], BBOX_CLAMP)
    pcx = dx * w + cx
    pcy = dy * h + cy
    pw = jnp.exp(dw) * w
    ph = jnp.exp(dh) * h
    return jnp.stack([pcx - 0.5 * pw, pcy - 0.5 * ph,
                      pcx + 0.5 * pw, pcy + 0.5 * ph], axis=1)


def _nms_body(bx_ref, sc_ref, lb_ref, ob_ref, os_ref, ol_ref):
    ncand = sc_ref.shape[2]
    x1 = bx_ref[0, 0:1, :]
    y1 = bx_ref[0, 1:2, :]
    x2 = bx_ref[0, 2:3, :]
    y2 = bx_ref[0, 3:4, :]
    scores = sc_ref[0]
    labf = lb_ref[0].astype(jnp.float32)
    offs = labf * (IMG + 1.0)
    nx1, ny1, nx2, ny2 = x1 + offs, y1 + offs, x2 + offs, y2 + offs
    area = (nx2 - nx1) * (ny2 - ny1)
    iota = lax.broadcasted_iota(jnp.int32, (1, ncand), 1)
    kiota = lax.broadcasted_iota(jnp.int32, (1, DETS), 1)
    zrow = jnp.zeros((1, DETS), jnp.float32)

    def step(i, carry):
        work, fs, fl, b1, b2, b3, b4 = carry
        mx = jnp.max(work)
        j = jnp.min(jnp.where(work == mx, iota, ncand))
        jm = iota == j

        def ext(row):
            return jnp.sum(jnp.where(jm, row, 0.0))

        jx1, jy1, jx2, jy2 = ext(nx1), ext(ny1), ext(nx2), ext(ny2)
        ja = (jx2 - jx1) * (jy2 - jy1)
        inter = (jnp.maximum(jnp.minimum(jx2, nx2) - jnp.maximum(jx1, nx1), 0.0)
                 * jnp.maximum(jnp.minimum(jy2, ny2) - jnp.maximum(jy1, ny1), 0.0))
        iou = inter / (ja + area - inter + 1e-9)
        im = kiota == i
        fs = jnp.where(im, ext(scores), fs)
        fl = jnp.where(im, ext(labf), fl)
        b1 = jnp.where(im, ext(x1), b1)
        b2 = jnp.where(im, ext(y1), b2)
        b3 = jnp.where(im, ext(x2), b3)
        b4 = jnp.where(im, ext(y2), b4)
        work = jnp.where(iou > NMS_THRESH, -jnp.inf, work)
        work = jnp.where(jm, -jnp.inf, work)
        return work, fs, fl, b1, b2, b3, b4

    _, fs, fl, b1, b2, b3, b4 = lax.fori_loop(
        0, DETS, step, (scores, zrow, zrow, zrow, zrow, zrow, zrow))

    valid = fs > SCORE_THRESH
    vf = valid.astype(jnp.float32)
    os_ref[0] = jnp.where(valid, fs, 0.0)
    ol_ref[0] = jnp.where(valid, fl, 0.0).astype(jnp.int32)
    ob_ref[0, 0:1, :] = b1 * vf
    ob_ref[0, 1:2, :] = b2 * vf
    ob_ref[0, 2:3, :] = b3 * vf
    ob_ref[0, 3:4, :] = b4 * vf


def _nms(boxes_t, scores, labels):
    ncand = scores.shape[1]
    ob, os_, ol = pl.pallas_call(
        _nms_body,
        grid=(B,),
        in_specs=[
            pl.BlockSpec((1, 4, ncand), lambda b: (b, 0, 0)),
            pl.BlockSpec((1, 1, ncand), lambda b: (b, 0, 0)),
            pl.BlockSpec((1, 1, ncand), lambda b: (b, 0, 0)),
        ],
        out_specs=[
            pl.BlockSpec((1, 4, DETS), lambda b: (b, 0, 0)),
            pl.BlockSpec((1, 1, DETS), lambda b: (b, 0, 0)),
            pl.BlockSpec((1, 1, DETS), lambda b: (b, 0, 0)),
        ],
        out_shape=[
            jax.ShapeDtypeStruct((B, 4, DETS), jnp.float32),
            jax.ShapeDtypeStruct((B, 1, DETS), jnp.float32),
            jax.ShapeDtypeStruct((B, 1, DETS), jnp.int32),
        ],
    )(boxes_t, scores[:, None, :], labels[:, None, :])
    return ob, os_[:, 0, :], ol[:, 0, :]


def kernel(cls_logits_l0, cls_logits_l1, cls_logits_l2, cls_logits_l3,
           bbox_reg_l0, bbox_reg_l1, bbox_reg_l2, bbox_reg_l3,
           anchors_l0, anchors_l1, anchors_l2, anchors_l3):
    logits = [cls_logits_l0, cls_logits_l1, cls_logits_l2, cls_logits_l3]
    regs = [bbox_reg_l0, bbox_reg_l1, bbox_reg_l2, bbox_reg_l3]
    ancs = [anchors_l0, anchors_l1, anchors_l2, anchors_l3]
    blks = [2048, 2048, 2048, 1536]

    m_list = [_anchor_max(logits[l], blks[l]) for l in range(4)]
    idx = _select(m_list)

    outs = []
    for b in range(B):
        all_b, all_s, all_l = [], [], []
        for l in range(4):
            a_sel = idx[b, l]
            glog = logits[l][b][a_sel]
            s = jax.nn.sigmoid(glog)
            sc = jnp.where(s > SCORE_THRESH, s, -1.0).reshape(-1)
            top_s, top_i = lax.top_k(sc, DETS)
            a_idx = a_sel[top_i // C]
            labels = top_i % C
            boxes = _decode(regs[l][b][a_idx], ancs[l][a_idx])
            boxes = jnp.clip(boxes, 0.0, IMG)
            all_b.append(boxes)
            all_s.append(top_s)
            all_l.append(labels)
        outs.append((jnp.concatenate(all_b, axis=0),
                     jnp.concatenate(all_s, axis=0),
                     jnp.concatenate(all_l, axis=0)))
    boxes_t = jnp.stack([o[0].T for o in outs], axis=0)
    scores = jnp.stack([o[1] for o in outs], axis=0)
    labels = jnp.stack([o[2] for o in outs], axis=0)
    ob, os_, ol = _nms(boxes_t, scores, labels)
    return jnp.swapaxes(ob, 1, 2), os_, ol


# speedup vs baseline: 2.3976x; 1.0232x over previous
import functools
import math

import jax
import jax.numpy as jnp
from jax import lax
from jax.experimental import pallas as pl
from jax.experimental.pallas import tpu as pltpu
from jax.experimental.pallas import tpu_sc as plsc

B = 2
C = 80
IMG = 800.0
SCORE_THRESH = 0.05
NMS_THRESH = 0.5
DETS = 100
BBOX_CLAMP = 4.135166556742356
K_ANC = 128
T_LOGIT = -math.log((1.0 - SCORE_THRESH) / SCORE_THRESH)


def _amax_body(n, a_blk, x_ref, o_ref):
    i = pl.program_id(1)
    x = x_ref[0]
    xm = jnp.max(jnp.where(x > T_LOGIT, x, -1e30), axis=1)
    m = jnp.where(xm > -1e29, 1.0 / (1.0 + jnp.exp(-xm)), -1.0)
    rows = i * a_blk + lax.broadcasted_iota(jnp.int32, (a_blk,), 0)
    m = jnp.where(rows < n, m, -2.0)
    o_ref[0, 0] = m


def _anchor_max(x, a_blk):
    n = x.shape[1]
    nb = pl.cdiv(n, a_blk)
    out = pl.pallas_call(
        functools.partial(_amax_body, n, a_blk),
        grid=(B, nb),
        in_specs=[pl.BlockSpec((1, a_blk, C), lambda b, i: (b, i, 0))],
        out_specs=pl.BlockSpec((1, 1, a_blk), lambda b, i: (b * nb + i, 0, 0)),
        out_shape=jax.ShapeDtypeStruct((B * nb, 1, a_blk), jnp.float32),
    )(x)
    return out.reshape(B, nb * a_blk)


_KEY_NEG1 = -1065353217
_KEY_005 = 1028443341
_KEY_HI = 1065353216
_BISECT_ITERS = 26


def _keys_of(m):
    bi = lax.bitcast_convert_type(m, jnp.int32)
    return jnp.where(bi >= 0, bi, jnp.int32(-2147483648) ^ (~bi))


def _tr(x, eye):
    if x.shape[1] == 1:
        return lax.dot_general(x, eye, (((0,), (0,)), ((), ())),
                               preferred_element_type=jnp.float32)
    return lax.dot_general(eye, x, (((1,), (1,)), ((), ())),
                           preferred_element_type=jnp.float32)


def _dot(a, b):
    return jnp.dot(a, b, preferred_element_type=jnp.float32)


def _select_body(rows_list, *refs):
    m_refs, idx_ref, kscr = refs[:4], refs[4], refs[5]
    offs, o = [], 0
    for r in rows_list:
        offs.extend([o, o + r])
        o += 2 * r
    tasks = [(l, b, rows_list[l], offs[2 * l + b])
             for l in range(4) for b in range(B)]
    for l, b, rows, off in tasks:
        kscr[pl.ds(off, rows), :] = _keys_of(m_refs[l][b])

    def cnt(off, rows, t):
        return jnp.sum((kscr[pl.ds(off, rows), :] > t).astype(jnp.int32))

    g05s = [cnt(off, rows, _KEY_005 - 1) for _, _, rows, off in tasks]

    def bis(_, c):
        los, his = c
        nlo, nhi = [], []
        for (l, b, rows, off), lo, hi in zip(tasks, los, his):
            mid = lo + (hi - lo) // 2
            big = cnt(off, rows, mid) >= K_ANC
            nlo.append(jnp.where(big, mid, lo))
            nhi.append(jnp.where(big, hi, mid))
        return tuple(nlo), tuple(nhi)

    init = (tuple(jnp.int32(_KEY_005 - 1) for _ in tasks),
            tuple(jnp.int32(_KEY_HI) for _ in tasks))
    _, his = lax.fori_loop(0, _BISECT_ITERS, bis, init)
    vs = [jnp.where(g05 >= K_ANC, hi, jnp.int32(_KEY_NEG1))
          for g05, hi in zip(g05s, his)]
    gs = [cnt(off, rows, v) for (_, _, rows, off), v in zip(tasks, vs)]

    ut128 = (lax.broadcasted_iota(jnp.int32, (128, 128), 0)
             < lax.broadcasted_iota(jnp.int32, (128, 128), 1)).astype(jnp.float32)
    lane_f = lax.broadcasted_iota(jnp.int32, (128, 128), 1).astype(jnp.float32)
    k_col = lax.broadcasted_iota(jnp.int32, (K_ANC, 1), 0).astype(jnp.float32)

    eyes, uts = {}, {}
    for r in set(rows_list):
        i0 = lax.broadcasted_iota(jnp.int32, (r, r), 0)
        i1 = lax.broadcasted_iota(jnp.int32, (r, r), 1)
        eyes[r] = (i0 == i1).astype(jnp.float32)
        uts[r] = (i0 < i1).astype(jnp.float32)

    for (l, b, rows, off), v, g in zip(tasks, vs, gs):
        keys = kscr[pl.ds(off, rows), :]
        strict = keys > v
        tie = keys == v
        tie_f = tie.astype(jnp.float32)
        p_tie = _dot(tie_f, ut128)
        tcnt_col = p_tie[:, 127:128] + tie_f[:, 127:128]
        tpref_row = _dot(_tr(tcnt_col, eyes[rows]), uts[rows])
        tie_rank = _tr(tpref_row, eyes[rows]) + p_tie
        g_f = g.astype(jnp.float32)
        sel = strict | (tie & (g_f + tie_rank < float(K_ANC)))
        sel_f = sel.astype(jnp.float32)
        p_sel = _dot(sel_f, ut128)
        scnt_col = p_sel[:, 127:128] + sel_f[:, 127:128]
        spref_row = _dot(_tr(scnt_col, eyes[rows]), uts[rows])
        rmask = (spref_row <= k_col).astype(jnp.float32)
        r_col = jnp.sum(rmask, axis=1, keepdims=True) - 1.0
        onehot_r = (lax.broadcasted_iota(jnp.int32, (K_ANC, rows), 1)
                    .astype(jnp.float32) == r_col).astype(jnp.float32)
        spref_at_k = _dot(onehot_r, _tr(spref_row, eyes[rows]))
        gmat = _dot(onehot_r, jnp.where(sel, p_sel, 1e9))
        q_col = k_col - spref_at_k
        c_col = jnp.sum(jnp.where(gmat == q_col, lane_f, 0.0),
                        axis=1, keepdims=True)
        idx_ref[b, l] = (r_col * 128.0 + c_col).astype(jnp.int32)


def _select(m_list):
    rows_list = [m.shape[1] // 128 for m in m_list]
    ms = [m.reshape(B, r, 128) for m, r in zip(m_list, rows_list)]
    total = 2 * sum(rows_list)
    out = pl.pallas_call(
        functools.partial(_select_body, rows_list),
        in_specs=[pl.BlockSpec(memory_space=pltpu.VMEM) for _ in ms],
        out_specs=pl.BlockSpec(memory_space=pltpu.VMEM),
        out_shape=jax.ShapeDtypeStruct((B, 4, K_ANC, 1), jnp.int32),
        scratch_shapes=[pltpu.VMEM((total, 128), jnp.int32)],
    )(*ms)
    return out.reshape(B, 4, K_ANC)


def _pair(logits, idx):
    glog = jnp.stack([
        jnp.stack([logits[l][b][idx[b, l]] for l in range(4)], axis=0)
        for b in range(B)], axis=0)
    s = jax.nn.sigmoid(glog)
    sc = jnp.where(s > SCORE_THRESH, s, -1.0).reshape(B * 4, K_ANC * C)
    top_s, top_i = lax.top_k(sc, DETS)
    j = top_i // C
    c = top_i % C
    return (top_s.reshape(B, 4, DETS), j.reshape(B, 4, DETS),
            c.reshape(B, 4, DETS))


def _decode(rel, anc):
    w = anc[:, 2] - anc[:, 0]
    h = anc[:, 3] - anc[:, 1]
    cx = anc[:, 0] + 0.5 * w
    cy = anc[:, 1] + 0.5 * h
    dx, dy = rel[:, 0], rel[:, 1]
    dw = jnp.minimum(rel[:, 2], BBOX_CLAMP)
    dh = jnp.minimum(rel[:, 3], BBOX_CLAMP)
    pcx = dx * w + cx
    pcy = dy * h + cy
    pw = jnp.exp(dw) * w
    ph = jnp.exp(dh) * h
    return jnp.stack([pcx - 0.5 * pw, pcy - 0.5 * ph,
                      pcx + 0.5 * pw, pcy + 0.5 * ph], axis=1)


def _nms_body(bx_ref, sc_ref, lb_ref, ob_ref, os_ref, ol_ref):
    ncand = sc_ref.shape[2]
    x1 = bx_ref[0, 0:1, :]
    y1 = bx_ref[0, 1:2, :]
    x2 = bx_ref[0, 2:3, :]
    y2 = bx_ref[0, 3:4, :]
    scores = sc_ref[0]
    labf = lb_ref[0].astype(jnp.float32)
    offs = labf * (IMG + 1.0)
    nx1, ny1, nx2, ny2 = x1 + offs, y1 + offs, x2 + offs, y2 + offs
    area = (nx2 - nx1) * (ny2 - ny1)
    iota = lax.broadcasted_iota(jnp.int32, (1, ncand), 1)
    kiota = lax.broadcasted_iota(jnp.int32, (1, DETS), 1)
    zrow = jnp.zeros((1, DETS), jnp.float32)

    def step(i, carry):
        work, fs, fl, b1, b2, b3, b4 = carry
        mx = jnp.max(work)
        j = jnp.min(jnp.where(work == mx, iota, ncand))
        jm = iota == j

        def ext(row):
            return jnp.sum(jnp.where(jm, row, 0.0))

        jx1, jy1, jx2, jy2 = ext(nx1), ext(ny1), ext(nx2), ext(ny2)
        ja = (jx2 - jx1) * (jy2 - jy1)
        inter = (jnp.maximum(jnp.minimum(jx2, nx2) - jnp.maximum(jx1, nx1), 0.0)
                 * jnp.maximum(jnp.minimum(jy2, ny2) - jnp.maximum(jy1, ny1), 0.0))
        iou = inter / (ja + area - inter + 1e-9)
        im = kiota == i
        fs = jnp.where(im, ext(scores), fs)
        fl = jnp.where(im, ext(labf), fl)
        b1 = jnp.where(im, ext(x1), b1)
        b2 = jnp.where(im, ext(y1), b2)
        b3 = jnp.where(im, ext(x2), b3)
        b4 = jnp.where(im, ext(y2), b4)
        work = jnp.where(iou > NMS_THRESH, -jnp.inf, work)
        work = jnp.where(jm, -jnp.inf, work)
        return work, fs, fl, b1, b2, b3, b4

    _, fs, fl, b1, b2, b3, b4 = lax.fori_loop(
        0, DETS, step, (scores, zrow, zrow, zrow, zrow, zrow, zrow))

    valid = fs > SCORE_THRESH
    vf = valid.astype(jnp.float32)
    os_ref[0] = jnp.where(valid, fs, 0.0)
    ol_ref[0] = jnp.where(valid, fl, 0.0).astype(jnp.int32)
    ob_ref[0, 0:1, :] = b1 * vf
    ob_ref[0, 1:2, :] = b2 * vf
    ob_ref[0, 2:3, :] = b3 * vf
    ob_ref[0, 3:4, :] = b4 * vf


def _nms(boxes_t, scores, labels):
    ncand = scores.shape[1]
    ob, os_, ol = pl.pallas_call(
        _nms_body,
        grid=(B,),
        in_specs=[
            pl.BlockSpec((1, 4, ncand), lambda b: (b, 0, 0)),
            pl.BlockSpec((1, 1, ncand), lambda b: (b, 0, 0)),
            pl.BlockSpec((1, 1, ncand), lambda b: (b, 0, 0)),
        ],
        out_specs=[
            pl.BlockSpec((1, 4, DETS), lambda b: (b, 0, 0)),
            pl.BlockSpec((1, 1, DETS), lambda b: (b, 0, 0)),
            pl.BlockSpec((1, 1, DETS), lambda b: (b, 0, 0)),
        ],
        out_shape=[
            jax.ShapeDtypeStruct((B, 4, DETS), jnp.float32),
            jax.ShapeDtypeStruct((B, 1, DETS), jnp.float32),
            jax.ShapeDtypeStruct((B, 1, DETS), jnp.int32),
        ],
    )(boxes_t, scores[:, None, :], labels[:, None, :])
    return ob, os_[:, 0, :], ol[:, 0, :]


def kernel(cls_logits_l0, cls_logits_l1, cls_logits_l2, cls_logits_l3,
           bbox_reg_l0, bbox_reg_l1, bbox_reg_l2, bbox_reg_l3,
           anchors_l0, anchors_l1, anchors_l2, anchors_l3):
    logits = [cls_logits_l0, cls_logits_l1, cls_logits_l2, cls_logits_l3]
    regs = [bbox_reg_l0, bbox_reg_l1, bbox_reg_l2, bbox_reg_l3]
    ancs = [anchors_l0, anchors_l1, anchors_l2, anchors_l3]
    blks = [2048, 2048, 2048, 1536]

    m_list = [_anchor_max(logits[l], blks[l]) for l in range(4)]
    idx = _select(m_list)
    ts, jr, cr = _pair(logits, idx)

    outs = []
    for b in range(B):
        all_b, all_s, all_l = [], [], []
        for l in range(4):
            top_s = ts[b, l]
            a_idx = idx[b, l][jr[b, l]]
            labels = cr[b, l]
            boxes = _decode(regs[l][b][a_idx], ancs[l][a_idx])
            boxes = jnp.clip(boxes, 0.0, IMG)
            all_b.append(boxes)
            all_s.append(top_s)
            all_l.append(labels)
        outs.append((jnp.concatenate(all_b, axis=0),
                     jnp.concatenate(all_s, axis=0),
                     jnp.concatenate(all_l, axis=0)))
    boxes_t = jnp.stack([o[0].T for o in outs], axis=0)
    scores = jnp.stack([o[1] for o in outs], axis=0)
    labels = jnp.stack([o[2] for o in outs], axis=0)
    ob, os_, ol = _nms(boxes_t, scores, labels)
    return jnp.swapaxes(ob, 1, 2), os_, ol


# speedup vs baseline: 2.3982x; 1.0003x over previous
import functools
import math

import jax
import jax.numpy as jnp
from jax import lax
from jax.experimental import pallas as pl
from jax.experimental.pallas import tpu as pltpu

B = 2
C = 80
IMG = 800.0
SCORE_THRESH = 0.05
NMS_THRESH = 0.5
DETS = 100
BBOX_CLAMP = 4.135166556742356
K_ANC = 128
T_LOGIT = -math.log((1.0 - SCORE_THRESH) / SCORE_THRESH)


def _amax_body(n, a_blk, x_ref, o_ref):
    i = pl.program_id(1)
    x = x_ref[0]
    xm = jnp.max(jnp.where(x > T_LOGIT, x, -1e30), axis=1)
    m = jnp.where(xm > -1e29, 1.0 / (1.0 + jnp.exp(-xm)), -1.0)
    rows = i * a_blk + lax.broadcasted_iota(jnp.int32, (a_blk,), 0)
    m = jnp.where(rows < n, m, -2.0)
    o_ref[0, 0] = m


def _anchor_max(x, a_blk):
    n = x.shape[1]
    nb = pl.cdiv(n, a_blk)
    out = pl.pallas_call(
        functools.partial(_amax_body, n, a_blk),
        grid=(B, nb),
        in_specs=[pl.BlockSpec((1, a_blk, C), lambda b, i: (b, i, 0))],
        out_specs=pl.BlockSpec((1, 1, a_blk), lambda b, i: (b * nb + i, 0, 0)),
        out_shape=jax.ShapeDtypeStruct((B * nb, 1, a_blk), jnp.float32),
    )(x)
    return out.reshape(B, nb * a_blk)


_KEY_NEG1 = -1065353217
_KEY_005 = 1028443341
_KEY_HI = 1065353216
_BISECT_ITERS = 26


def _keys_of(m):
    bi = lax.bitcast_convert_type(m, jnp.int32)
    return jnp.where(bi >= 0, bi, jnp.int32(-2147483648) ^ (~bi))


def _tr(x, eye):
    if x.shape[1] == 1:
        return lax.dot_general(x, eye, (((0,), (0,)), ((), ())),
                               preferred_element_type=jnp.float32)
    return lax.dot_general(eye, x, (((1,), (1,)), ((), ())),
                           preferred_element_type=jnp.float32)


def _dot(a, b):
    return jnp.dot(a, b, preferred_element_type=jnp.float32)


def _select_body(rows_list, *refs):
    m_refs, idx_ref, kscr = refs[:4], refs[4], refs[5]
    offs, o = [], 0
    for r in rows_list:
        offs.extend([o, o + r])
        o += 2 * r
    tasks = [(l, b, rows_list[l], offs[2 * l + b])
             for l in range(4) for b in range(B)]
    for l, b, rows, off in tasks:
        kscr[pl.ds(off, rows), :] = _keys_of(m_refs[l][b])

    def cnt(off, rows, t):
        return jnp.sum((kscr[pl.ds(off, rows), :] > t).astype(jnp.int32))

    g05s = [cnt(off, rows, _KEY_005 - 1) for _, _, rows, off in tasks]

    def bis(_, c):
        los, his = c
        nlo, nhi = [], []
        for (l, b, rows, off), lo, hi in zip(tasks, los, his):
            mid = lo + (hi - lo) // 2
            big = cnt(off, rows, mid) >= K_ANC
            nlo.append(jnp.where(big, mid, lo))
            nhi.append(jnp.where(big, hi, mid))
        return tuple(nlo), tuple(nhi)

    init = (tuple(jnp.int32(_KEY_005 - 1) for _ in tasks),
            tuple(jnp.int32(_KEY_HI) for _ in tasks))
    _, his = lax.fori_loop(0, _BISECT_ITERS, bis, init)
    vs = [jnp.where(g05 >= K_ANC, hi, jnp.int32(_KEY_NEG1))
          for g05, hi in zip(g05s, his)]
    gs = [cnt(off, rows, v) for (_, _, rows, off), v in zip(tasks, vs)]

    ut128 = (lax.broadcasted_iota(jnp.int32, (128, 128), 0)
             < lax.broadcasted_iota(jnp.int32, (128, 128), 1)).astype(jnp.float32)
    lane_f = lax.broadcasted_iota(jnp.int32, (128, 128), 1).astype(jnp.float32)
    k_col = lax.broadcasted_iota(jnp.int32, (K_ANC, 1), 0).astype(jnp.float32)

    eyes, uts = {}, {}
    for r in set(rows_list):
        i0 = lax.broadcasted_iota(jnp.int32, (r, r), 0)
        i1 = lax.broadcasted_iota(jnp.int32, (r, r), 1)
        eyes[r] = (i0 == i1).astype(jnp.float32)
        uts[r] = (i0 < i1).astype(jnp.float32)

    for (l, b, rows, off), v, g in zip(tasks, vs, gs):
        keys = kscr[pl.ds(off, rows), :]
        strict = keys > v
        tie = keys == v
        tie_f = tie.astype(jnp.float32)
        p_tie = _dot(tie_f, ut128)
        tcnt_col = p_tie[:, 127:128] + tie_f[:, 127:128]
        tpref_row = _dot(_tr(tcnt_col, eyes[rows]), uts[rows])
        tie_rank = _tr(tpref_row, eyes[rows]) + p_tie
        g_f = g.astype(jnp.float32)
        sel = strict | (tie & (g_f + tie_rank < float(K_ANC)))
        sel_f = sel.astype(jnp.float32)
        p_sel = _dot(sel_f, ut128)
        scnt_col = p_sel[:, 127:128] + sel_f[:, 127:128]
        spref_row = _dot(_tr(scnt_col, eyes[rows]), uts[rows])
        rmask = (spref_row <= k_col).astype(jnp.float32)
        r_col = jnp.sum(rmask, axis=1, keepdims=True) - 1.0
        onehot_r = (lax.broadcasted_iota(jnp.int32, (K_ANC, rows), 1)
                    .astype(jnp.float32) == r_col).astype(jnp.float32)
        spref_at_k = _dot(onehot_r, _tr(spref_row, eyes[rows]))
        gmat = _dot(onehot_r, jnp.where(sel, p_sel, 1e9))
        q_col = k_col - spref_at_k
        c_col = jnp.sum(jnp.where(gmat == q_col, lane_f, 0.0),
                        axis=1, keepdims=True)
        idx_ref[b, l] = (r_col * 128.0 + c_col).astype(jnp.int32)


def _select(m_list):
    rows_list = [m.shape[1] // 128 for m in m_list]
    ms = [m.reshape(B, r, 128) for m, r in zip(m_list, rows_list)]
    total = 2 * sum(rows_list)
    out = pl.pallas_call(
        functools.partial(_select_body, rows_list),
        in_specs=[pl.BlockSpec(memory_space=pltpu.VMEM) for _ in ms],
        out_specs=pl.BlockSpec(memory_space=pltpu.VMEM),
        out_shape=jax.ShapeDtypeStruct((B, 4, K_ANC, 1), jnp.int32),
        scratch_shapes=[pltpu.VMEM((total, 128), jnp.int32)],
    )(*ms)
    return out.reshape(B, 4, K_ANC)


def _pair(logits, idx):
    glog = jnp.stack([
        jnp.stack([logits[l][b][idx[b, l]] for l in range(4)], axis=0)
        for b in range(B)], axis=0)
    s = jax.nn.sigmoid(glog)
    sc = jnp.where(s > SCORE_THRESH, s, -1.0).reshape(B * 4, K_ANC * C)
    top_s, top_i = lax.top_k(sc, DETS)
    j = top_i // C
    c = top_i % C
    return (top_s.reshape(B, 4, DETS), j.reshape(B, 4, DETS),
            c.reshape(B, 4, DETS))


def _decode(rel, anc):
    w = anc[:, 2] - anc[:, 0]
    h = anc[:, 3] - anc[:, 1]
    cx = anc[:, 0] + 0.5 * w
    cy = anc[:, 1] + 0.5 * h
    dx, dy = rel[:, 0], rel[:, 1]
    dw = jnp.minimum(rel[:, 2], BBOX_CLAMP)
    dh = jnp.minimum(rel[:, 3], BBOX_CLAMP)
    pcx = dx * w + cx
    pcy = dy * h + cy
    pw = jnp.exp(dw) * w
    ph = jnp.exp(dh) * h
    return jnp.stack([pcx - 0.5 * pw, pcy - 0.5 * ph,
                      pcx + 0.5 * pw, pcy + 0.5 * ph], axis=1)


def _nms_body(bx_ref, sc_ref, lb_ref, ob_ref, os_ref, ol_ref):
    ncand = sc_ref.shape[2]
    x1 = bx_ref[0, 0:1, :]
    y1 = bx_ref[0, 1:2, :]
    x2 = bx_ref[0, 2:3, :]
    y2 = bx_ref[0, 3:4, :]
    scores = sc_ref[0]
    labf = lb_ref[0].astype(jnp.float32)
    offs = labf * (IMG + 1.0)
    nx1, ny1, nx2, ny2 = x1 + offs, y1 + offs, x2 + offs, y2 + offs
    area = (nx2 - nx1) * (ny2 - ny1)
    iota = lax.broadcasted_iota(jnp.int32, (1, ncand), 1)
    kiota = lax.broadcasted_iota(jnp.int32, (1, DETS), 1)
    zrow = jnp.zeros((1, DETS), jnp.float32)

    def step(i, carry):
        work, fs, fl, b1, b2, b3, b4 = carry
        mx = jnp.max(work)
        j = jnp.min(jnp.where(work == mx, iota, ncand))
        jm = iota == j

        def ext(row):
            return jnp.sum(jnp.where(jm, row, 0.0))

        jx1, jy1, jx2, jy2 = ext(nx1), ext(ny1), ext(nx2), ext(ny2)
        ja = (jx2 - jx1) * (jy2 - jy1)
        inter = (jnp.maximum(jnp.minimum(jx2, nx2) - jnp.maximum(jx1, nx1), 0.0)
                 * jnp.maximum(jnp.minimum(jy2, ny2) - jnp.maximum(jy1, ny1), 0.0))
        iou = inter / (ja + area - inter + 1e-9)
        im = kiota == i
        fs = jnp.where(im, ext(scores), fs)
        fl = jnp.where(im, ext(labf), fl)
        b1 = jnp.where(im, ext(x1), b1)
        b2 = jnp.where(im, ext(y1), b2)
        b3 = jnp.where(im, ext(x2), b3)
        b4 = jnp.where(im, ext(y2), b4)
        work = jnp.where(iou > NMS_THRESH, -jnp.inf, work)
        work = jnp.where(jm, -jnp.inf, work)
        return work, fs, fl, b1, b2, b3, b4

    _, fs, fl, b1, b2, b3, b4 = lax.fori_loop(
        0, DETS, step, (scores, zrow, zrow, zrow, zrow, zrow, zrow))

    valid = fs > SCORE_THRESH
    vf = valid.astype(jnp.float32)
    os_ref[0] = jnp.where(valid, fs, 0.0)
    ol_ref[0] = jnp.where(valid, fl, 0.0).astype(jnp.int32)
    ob_ref[0, 0:1, :] = b1 * vf
    ob_ref[0, 1:2, :] = b2 * vf
    ob_ref[0, 2:3, :] = b3 * vf
    ob_ref[0, 3:4, :] = b4 * vf


def _nms(boxes_t, scores, labels):
    ncand = scores.shape[1]
    ob, os_, ol = pl.pallas_call(
        _nms_body,
        grid=(B,),
        in_specs=[
            pl.BlockSpec((1, 4, ncand), lambda b: (b, 0, 0)),
            pl.BlockSpec((1, 1, ncand), lambda b: (b, 0, 0)),
            pl.BlockSpec((1, 1, ncand), lambda b: (b, 0, 0)),
        ],
        out_specs=[
            pl.BlockSpec((1, 4, DETS), lambda b: (b, 0, 0)),
            pl.BlockSpec((1, 1, DETS), lambda b: (b, 0, 0)),
            pl.BlockSpec((1, 1, DETS), lambda b: (b, 0, 0)),
        ],
        out_shape=[
            jax.ShapeDtypeStruct((B, 4, DETS), jnp.float32),
            jax.ShapeDtypeStruct((B, 1, DETS), jnp.float32),
            jax.ShapeDtypeStruct((B, 1, DETS), jnp.int32),
        ],
    )(boxes_t, scores[:, None, :], labels[:, None, :])
    return ob, os_[:, 0, :], ol[:, 0, :]


def kernel(cls_logits_l0, cls_logits_l1, cls_logits_l2, cls_logits_l3,
           bbox_reg_l0, bbox_reg_l1, bbox_reg_l2, bbox_reg_l3,
           anchors_l0, anchors_l1, anchors_l2, anchors_l3):
    logits = [cls_logits_l0, cls_logits_l1, cls_logits_l2, cls_logits_l3]
    regs = [bbox_reg_l0, bbox_reg_l1, bbox_reg_l2, bbox_reg_l3]
    ancs = [anchors_l0, anchors_l1, anchors_l2, anchors_l3]
    blks = [2048, 2048, 2048, 1536]

    m_list = [_anchor_max(logits[l], blks[l]) for l in range(4)]
    idx = _select(m_list)
    ts, jr, cr = _pair(logits, idx)

    outs = []
    for b in range(B):
        all_b, all_s, all_l = [], [], []
        for l in range(4):
            top_s = ts[b, l]
            a_idx = idx[b, l][jr[b, l]]
            labels = cr[b, l]
            boxes = _decode(regs[l][b][a_idx], ancs[l][a_idx])
            boxes = jnp.clip(boxes, 0.0, IMG)
            all_b.append(boxes)
            all_s.append(top_s)
            all_l.append(labels)
        outs.append((jnp.concatenate(all_b, axis=0),
                     jnp.concatenate(all_s, axis=0),
                     jnp.concatenate(all_l, axis=0)))
    boxes_t = jnp.stack([o[0].T for o in outs], axis=0)
    scores = jnp.stack([o[1] for o in outs], axis=0)
    labels = jnp.stack([o[2] for o in outs], axis=0)
    ob, os_, ol = _nms(boxes_t, scores, labels)
    return jnp.swapaxes(ob, 1, 2), os_, ol
